# baseline (speedup 1.0000x reference)
import jax
import jax.numpy as jnp
from jax.experimental import pallas as pl
from jax.experimental.pallas import tpu as pltpu

_B, _N, _D = 8, 512, 320
_E = 16
_K = 8192
_QB = 1024
_BN = _B * _N
_GRID = _BN // _QB


def _rpq_kernel(x_ref, rp_ref, cbt_ref, out_ref):
    xq = x_ref[...].reshape(_QB, _D)
    proj = jnp.dot(xq, rp_ref[...], preferred_element_type=jnp.float32)
    xnorm = jnp.sqrt(jnp.sum(proj * proj, axis=1, keepdims=True))
    xn = proj / jnp.maximum(xnorm, 1e-12)

    cbt = cbt_ref[...]
    cnorm = jnp.sqrt(jnp.sum(cbt * cbt, axis=0, keepdims=True))
    cn = cbt / jnp.maximum(cnorm, 1e-12)
    c2 = jnp.sum(cn * cn, axis=0, keepdims=True)

    xa = jnp.concatenate([xn, jnp.ones((xn.shape[0], 1), jnp.float32)], axis=1)
    cnb = jnp.concatenate([cn, -0.5 * c2], axis=0)
    _C = 7168
    s1 = jnp.dot(xa, cnb[:, :_C], preferred_element_type=jnp.float32)
    s2 = jnp.dot(xa, cnb[:, _C:], preferred_element_type=jnp.float32)
    m1 = jnp.max(s1, axis=1, keepdims=True)
    l1 = jax.lax.broadcasted_iota(jnp.int32, s1.shape, 1)
    e1 = jnp.min(jnp.where(s1 == m1, l1, _K), axis=1)
    m2 = jnp.max(s2, axis=1, keepdims=True)
    l2 = jax.lax.broadcasted_iota(jnp.int32, s2.shape, 1) + _C
    e2 = jnp.min(jnp.where(s2 == m2, l2, _K), axis=1)
    arg = jnp.where(m2[:, 0] > m1[:, 0], e2, e1)
    out_ref[0, 0, :] = arg.astype(jnp.int32)


def kernel(x, random_projection, codebook):
    cbt = codebook.T

    out = pl.pallas_call(
        _rpq_kernel,
        grid=(_GRID,),
        in_specs=[
            pl.BlockSpec((_QB // _N, _N, _D), lambda i: (i, 0, 0)),
            pl.BlockSpec((_D, _E), lambda i: (0, 0)),
            pl.BlockSpec((_E, _K), lambda i: (0, 0)),
        ],
        out_specs=pl.BlockSpec((1, 1, _QB), lambda i: (i, 0, 0)),
        out_shape=jax.ShapeDtypeStruct((_GRID, 1, _QB), jnp.int32),
        compiler_params=pltpu.CompilerParams(
            dimension_semantics=("parallel",),
            allow_input_fusion=[False, False, True],
        ),
    )(x, random_projection, cbt)
    return out.reshape(_B, _N)


# speedup vs baseline: 1.5020x; 1.5020x over previous
import jax
import jax.numpy as jnp
from jax.experimental import pallas as pl
from jax.experimental.pallas import tpu as pltpu

_B, _N, _D = 8, 512, 320
_E = 16
_K = 8192
_QB = 1024
_BN = _B * _N
_GRID = _BN // _QB


def _rpq_kernel(x_ref, rp_ref, cbt_ref, out_ref):
    xq = x_ref[...].reshape(_QB, _D)
    proj = jnp.dot(xq, rp_ref[...], preferred_element_type=jnp.float32)
    xnorm = jnp.sqrt(jnp.sum(proj * proj, axis=1, keepdims=True))
    xn = proj / jnp.maximum(xnorm, 1e-12)

    cbt = cbt_ref[...]
    cnorm = jnp.sqrt(jnp.sum(cbt * cbt, axis=0, keepdims=True))
    cn = cbt / jnp.maximum(cnorm, 1e-12)
    c2 = jnp.sum(cn * cn, axis=0, keepdims=True)

    xa = jnp.concatenate([xn, jnp.ones((xn.shape[0], 1), jnp.float32)], axis=1)
    cnb = jnp.concatenate([cn, -0.5 * c2], axis=0)
    scores = jnp.dot(xa, cnb, preferred_element_type=jnp.float32)

    arg = jnp.argmax(scores, axis=1)
    out_ref[0, 0, :] = arg.astype(jnp.int32)


def kernel(x, random_projection, codebook):
    cbt = codebook.T

    out = pl.pallas_call(
        _rpq_kernel,
        grid=(_GRID,),
        in_specs=[
            pl.BlockSpec((_QB // _N, _N, _D), lambda i: (i, 0, 0)),
            pl.BlockSpec((_D, _E), lambda i: (0, 0)),
            pl.BlockSpec((_E, _K), lambda i: (0, 0)),
        ],
        out_specs=pl.BlockSpec((1, 1, _QB), lambda i: (i, 0, 0)),
        out_shape=jax.ShapeDtypeStruct((_GRID, 1, _QB), jnp.int32),
        compiler_params=pltpu.CompilerParams(
            dimension_semantics=("parallel",),
            allow_input_fusion=[False, False, True],
        ),
    )(x, random_projection, cbt)
    return out.reshape(_B, _N)
